# baseline (speedup 1.0000x reference)
import functools

import jax
import jax.numpy as jnp
from jax import lax
from jax.experimental import pallas as pl
from jax.experimental.pallas import tpu as pltpu
from jax.experimental.pallas import tpu_sc as plsc

_B = 4096
_L = 200
_EMBED = 64
_NC = 2
_NS = 16
_NW = _NC * _NS
_DPW = _B // _NW
_C0 = 128
_C1 = _L - _C0


def _pool_body(x_hbm, table_hbm, out_hbm, idx_v, rows_a, rows_b, out_v, sem):
    wid = lax.axis_index("s") * _NC + lax.axis_index("c")
    base = wid * _DPW

    pltpu.sync_copy(x_hbm.at[pl.ds(base, _DPW)], idx_v)

    def doc_body(d, _):
        cp_a = pltpu.make_async_copy(
            table_hbm.at[idx_v.at[d, pl.ds(0, _C0)]], rows_a, sem)
        cp_b = pltpu.make_async_copy(
            table_hbm.at[idx_v.at[d, pl.ds(_C0, _C1)]], rows_b, sem)
        cp_a.start()
        cp_b.start()
        cp_a.wait()
        cp_b.wait()

        def row_body_a(i, acc):
            return tuple(
                acc[c] + rows_a[i, pl.ds(16 * c, 16)] for c in range(4))

        def row_body_b(i, acc):
            return tuple(
                acc[c] + rows_b[i, pl.ds(16 * c, 16)] for c in range(4))

        zero = jnp.zeros((16,), jnp.float32)
        acc = lax.fori_loop(0, _C0, row_body_a, (zero, zero, zero, zero))
        acc = lax.fori_loop(0, _C1, row_body_b, acc)
        for c in range(4):
            out_v[d, pl.ds(16 * c, 16)] = acc[c]
        return 0

    lax.fori_loop(0, _DPW, doc_body, 0)

    pltpu.sync_copy(out_v, out_hbm.at[pl.ds(base, _DPW)])


@functools.partial(jax.jit, static_argnames=())
def _pool(x, table):
    mesh = plsc.VectorSubcoreMesh(core_axis_name="c", subcore_axis_name="s")
    f = pl.kernel(
        _pool_body,
        out_type=jax.ShapeDtypeStruct((_B, _EMBED), jnp.float32),
        mesh=mesh,
        scratch_types=[
            pltpu.VMEM((_DPW, _L), jnp.int32),
            pltpu.VMEM((_C0, _EMBED), jnp.float32),
            pltpu.VMEM((_C1, _EMBED), jnp.float32),
            pltpu.VMEM((_DPW, _EMBED), jnp.float32),
            pltpu.SemaphoreType.DMA,
        ],
        compiler_params=pltpu.CompilerParams(use_tc_tiling_on_sc=False),
    )
    return f(x, table)


def _mlp_body(s_ref, w1_ref, b1_ref, w2_ref, b2_ref, o_ref):
    s = s_ref[...]
    norm = jnp.sqrt(jnp.sum(s * s, axis=1, keepdims=True))
    ns = s / jnp.maximum(norm, 1e-12)
    h = lax.dot_general(ns, w1_ref[...], (((1,), (1,)), ((), ())),
                        preferred_element_type=jnp.float32)
    h = jnp.maximum(h + b1_ref[...], 0.0)
    o = lax.dot_general(h, w2_ref[...], (((1,), (1,)), ((), ())),
                        preferred_element_type=jnp.float32)
    o = o + b2_ref[...]
    m = jnp.max(o, axis=1, keepdims=True)
    e = jnp.exp(o - m)
    o_ref[...] = e / jnp.sum(e, axis=1, keepdims=True)


def _mlp(s, W1, b1, W2, b2):
    blk = 512
    grid = _B // blk
    return pl.pallas_call(
        _mlp_body,
        grid=(grid,),
        in_specs=[
            pl.BlockSpec((blk, _EMBED), lambda i: (i, 0)),
            pl.BlockSpec(W1.shape, lambda i: (0, 0)),
            pl.BlockSpec((1, W1.shape[0]), lambda i: (0, 0)),
            pl.BlockSpec(W2.shape, lambda i: (0, 0)),
            pl.BlockSpec((1, W2.shape[0]), lambda i: (0, 0)),
        ],
        out_specs=pl.BlockSpec((blk, _EMBED), lambda i: (i, 0)),
        out_shape=jax.ShapeDtypeStruct((_B, _EMBED), jnp.float32),
    )(s, W1, b1, W2, b2)


def kernel(x, table, W1, b1, W2, b2):
    x = x.astype(jnp.int32)
    s = _pool(x, table)
    return _mlp(s, W1, b1.reshape(1, -1), W2, b2.reshape(1, -1))


# speedup vs baseline: 1.1680x; 1.1680x over previous
import functools

import jax
import jax.numpy as jnp
from jax import lax
from jax.experimental import pallas as pl
from jax.experimental.pallas import tpu as pltpu
from jax.experimental.pallas import tpu_sc as plsc

_B = 4096
_L = 200
_EMBED = 64
_NC = 2
_NS = 16
_NW = _NC * _NS
_DPW = _B // _NW
_C0 = 128
_C1 = _L - _C0


def _pool_body(x_hbm, table_hbm, out_hbm, idx_v, rows0, rows1, out_v,
               sem0, sem1):
    wid = lax.axis_index("s") * _NC + lax.axis_index("c")
    base = wid * _DPW

    pltpu.sync_copy(x_hbm.at[pl.ds(base, _DPW)], idx_v)

    def start(d, rows, sem):
        pltpu.make_async_copy(
            table_hbm.at[idx_v.at[d, pl.ds(0, _C0)]],
            rows.at[pl.ds(0, _C0)], sem).start()
        pltpu.make_async_copy(
            table_hbm.at[idx_v.at[d, pl.ds(_C0, _C1)]],
            rows.at[pl.ds(_C0, _C1)], sem).start()

    def wait(rows, sem):
        pltpu.make_async_copy(table_hbm.at[pl.ds(0, _L)], rows, sem).wait()

    def accum(rows, d):
        def row_body(i, acc):
            return tuple(
                acc[c] + rows[i, pl.ds(16 * c, 16)] for c in range(4))

        zero = jnp.zeros((16,), jnp.float32)
        acc = lax.fori_loop(0, _L, row_body, (zero, zero, zero, zero),
                            unroll=8)
        for c in range(4):
            out_v[d, pl.ds(16 * c, 16)] = acc[c]

    start(0, rows0, sem0)

    def body(t, _):
        d0 = 2 * t
        d1 = d0 + 1
        start(d1, rows1, sem1)
        wait(rows0, sem0)
        accum(rows0, d0)
        d2 = jnp.minimum(d0 + 2, _DPW - 1)
        start(d2, rows0, sem0)
        wait(rows1, sem1)
        accum(rows1, d1)
        return 0

    lax.fori_loop(0, _DPW // 2, body, 0)
    wait(rows0, sem0)

    pltpu.sync_copy(out_v, out_hbm.at[pl.ds(base, _DPW)])


@functools.partial(jax.jit, static_argnames=())
def _pool(x, table):
    mesh = plsc.VectorSubcoreMesh(core_axis_name="c", subcore_axis_name="s")
    f = pl.kernel(
        _pool_body,
        out_type=jax.ShapeDtypeStruct((_B, _EMBED), jnp.float32),
        mesh=mesh,
        scratch_types=[
            pltpu.VMEM((_DPW, _L), jnp.int32),
            pltpu.VMEM((_L, _EMBED), jnp.float32),
            pltpu.VMEM((_L, _EMBED), jnp.float32),
            pltpu.VMEM((_DPW, _EMBED), jnp.float32),
            pltpu.SemaphoreType.DMA,
            pltpu.SemaphoreType.DMA,
        ],
        compiler_params=pltpu.CompilerParams(use_tc_tiling_on_sc=False),
    )
    return f(x, table)


def _mlp_body(s_ref, w1_ref, b1_ref, w2_ref, b2_ref, o_ref):
    s = s_ref[...]
    norm = jnp.sqrt(jnp.sum(s * s, axis=1, keepdims=True))
    ns = s / jnp.maximum(norm, 1e-12)
    h = lax.dot_general(ns, w1_ref[...], (((1,), (1,)), ((), ())),
                        preferred_element_type=jnp.float32)
    h = jnp.maximum(h + b1_ref[...], 0.0)
    o = lax.dot_general(h, w2_ref[...], (((1,), (1,)), ((), ())),
                        preferred_element_type=jnp.float32)
    o = o + b2_ref[...]
    m = jnp.max(o, axis=1, keepdims=True)
    e = jnp.exp(o - m)
    o_ref[...] = e / jnp.sum(e, axis=1, keepdims=True)


def _mlp(s, W1, b1, W2, b2):
    blk = 512
    grid = _B // blk
    return pl.pallas_call(
        _mlp_body,
        grid=(grid,),
        in_specs=[
            pl.BlockSpec((blk, _EMBED), lambda i: (i, 0)),
            pl.BlockSpec(W1.shape, lambda i: (0, 0)),
            pl.BlockSpec((1, W1.shape[0]), lambda i: (0, 0)),
            pl.BlockSpec(W2.shape, lambda i: (0, 0)),
            pl.BlockSpec((1, W2.shape[0]), lambda i: (0, 0)),
        ],
        out_specs=pl.BlockSpec((blk, _EMBED), lambda i: (i, 0)),
        out_shape=jax.ShapeDtypeStruct((_B, _EMBED), jnp.float32),
    )(s, W1, b1, W2, b2)


def kernel(x, table, W1, b1, W2, b2):
    x = x.astype(jnp.int32)
    s = _pool(x, table)
    return _mlp(s, W1, b1.reshape(1, -1), W2, b2.reshape(1, -1))
